# baseline (speedup 1.0000x reference)
import functools

import jax
import jax.numpy as jnp
from jax import lax
from jax.experimental import pallas as pl
from jax.experimental.pallas import tpu as pltpu
from jax.experimental.pallas import tpu_sc as plsc

_L = 128


@functools.lru_cache(maxsize=None)
def _make_gather_kernel(B, D, NC, NS):
    NW = NC * NS
    b_per_w = B // NW
    n_idx_rows = b_per_w * D // _L
    mesh = plsc.VectorSubcoreMesh(core_axis_name="c", subcore_axis_name="s")

    @functools.partial(
        pl.kernel,
        mesh=mesh,
        out_type=jax.ShapeDtypeStruct((B * D,), jnp.float32),
        scratch_types=[
            pltpu.VMEM((b_per_w,), jnp.int32),
            pltpu.VMEM((b_per_w * D,), jnp.int32),
            pltpu.VMEM((b_per_w * D,), jnp.float32),
            pltpu.SemaphoreType.DMA,
        ],
    )
    def gather_scale(idx_hbm, tab_hbm, out_hbm, ridx_v, eidx_v, rows_v, sem):
        wid = lax.axis_index("s") * NC + lax.axis_index("c")
        base = wid * b_per_w
        pltpu.sync_copy(idx_hbm.at[pl.ds(base, b_per_w)], ridx_v)
        lanes = lax.iota(jnp.int32, 16)

        def build_group(g, carry):
            uv = ridx_v[pl.ds(g * 16, 16)] * D
            for s in range(16):
                r = g * 16 + s
                u0 = uv[s]
                for k in range(D // 16):
                    f = r * D + k * 16
                    eidx_v[pl.ds(f, 16)] = u0 + k * 16 + lanes
            return carry

        lax.fori_loop(0, b_per_w // 16, build_group, 0)

        pltpu.async_copy(tab_hbm.at[eidx_v], rows_v, sem).wait()

        def scale_row(r, carry):
            for k in range(8):
                sl = pl.ds(r * _L + k * 16, 16)
                rows_v[sl] = rows_v[sl] * 0.25
            return carry

        lax.fori_loop(0, b_per_w * D // _L, scale_row, 0)
        pltpu.sync_copy(rows_v, out_hbm.at[pl.ds(base * D, b_per_w * D)])

    return gather_scale


def kernel(users, items, user_table, item_table, edge_user, edge_item):
    B = users.shape[0]
    N, D = user_table.shape
    info = plsc.get_sparse_core_info()
    fn = _make_gather_kernel(B, D, info.num_cores, info.num_subcores)
    out_u = fn(users, user_table.reshape(-1))
    out_i = fn(items, item_table.reshape(-1))
    return out_u.reshape(B, D), out_i.reshape(B, D)


# speedup vs baseline: 1.6806x; 1.6806x over previous
import functools

import jax
import jax.numpy as jnp
from jax import lax
from jax.experimental import pallas as pl
from jax.experimental.pallas import tpu as pltpu
from jax.experimental.pallas import tpu_sc as plsc

_CHUNK = 128
_DP = 128


@functools.lru_cache(maxsize=None)
def _make_gather_kernel(B, D, NC, NS):
    NW = NC * NS
    b_per_w = B // NW
    n_chunks = b_per_w // _CHUNK
    mesh = plsc.VectorSubcoreMesh(core_axis_name="c", subcore_axis_name="s")

    @functools.partial(
        pl.kernel,
        mesh=mesh,
        out_type=jax.ShapeDtypeStruct((B, _DP), jnp.float32),
        scratch_types=[
            pltpu.VMEM((n_chunks, _CHUNK), jnp.int32),
            pltpu.VMEM((b_per_w, _DP), jnp.float32),
            pltpu.SemaphoreType.DMA,
        ],
    )
    def gather_scale(idx_hbm, tab_hbm, out_hbm, idx_v, rows_v, sem):
        wid = lax.axis_index("s") * NC + lax.axis_index("c")
        base = wid * b_per_w
        for j in range(n_chunks):
            pltpu.sync_copy(idx_hbm.at[pl.ds(base + j * _CHUNK, _CHUNK)],
                            idx_v.at[j])
        copies = [
            pltpu.async_copy(
                tab_hbm.at[idx_v.at[j]],
                rows_v.at[pl.ds(j * _CHUNK, _CHUNK)], sem)
            for j in range(n_chunks)
        ]
        for c in copies:
            c.wait()

        def scale_row(r, carry):
            for k in range(D // 16):
                sl = pl.ds(k * 16, 16)
                rows_v[r, sl] = rows_v[r, sl] * 0.25
            return carry

        lax.fori_loop(0, b_per_w, scale_row, 0)
        pltpu.sync_copy(rows_v, out_hbm.at[pl.ds(base, b_per_w)])

    return gather_scale


def kernel(users, items, user_table, item_table, edge_user, edge_item):
    B = users.shape[0]
    N, D = user_table.shape
    info = plsc.get_sparse_core_info()
    fn = _make_gather_kernel(B, D, info.num_cores, info.num_subcores)
    utp = jnp.pad(user_table, ((0, 0), (0, _DP - D)))
    itp = jnp.pad(item_table, ((0, 0), (0, _DP - D)))
    out_u = fn(users, utp)
    out_i = fn(items, itp)
    return out_u[:, :D], out_i[:, :D]


# speedup vs baseline: 1.6946x; 1.0083x over previous
import functools

import jax
import jax.numpy as jnp
from jax import lax
from jax.experimental import pallas as pl
from jax.experimental.pallas import tpu as pltpu
from jax.experimental.pallas import tpu_sc as plsc

_CHUNK = 128
_DP = 128


@functools.lru_cache(maxsize=None)
def _make_gather_kernel(B, D, NC, NS):
    NW = NC * NS
    b_per_w = B // NW
    n_chunks = b_per_w // _CHUNK
    mesh = plsc.VectorSubcoreMesh(core_axis_name="c", subcore_axis_name="s")

    @functools.partial(
        pl.kernel,
        mesh=mesh,
        out_type=jax.ShapeDtypeStruct((B, _DP), jnp.float32),
        scratch_types=[
            pltpu.VMEM((n_chunks, _CHUNK), jnp.int32),
            pltpu.VMEM((b_per_w, _DP), jnp.float32),
            pltpu.SemaphoreType.DMA,
        ],
    )
    def gather_scale(idx_hbm, tab_hbm, out_hbm, idx_v, rows_v, sem):
        wid = lax.axis_index("s") * NC + lax.axis_index("c")
        base = wid * b_per_w
        for j in range(n_chunks):
            pltpu.sync_copy(idx_hbm.at[pl.ds(base + j * _CHUNK, _CHUNK)],
                            idx_v.at[j])
        copies = [
            pltpu.async_copy(
                tab_hbm.at[idx_v.at[j]],
                rows_v.at[pl.ds(j * _CHUNK, _CHUNK)], sem)
            for j in range(n_chunks)
        ]
        for j, c in enumerate(copies):
            c.wait()

            def scale_row(r, carry, j=j):
                for k in range(D // 16):
                    sl = pl.ds(k * 16, 16)
                    rows_v[j * _CHUNK + r, sl] = (
                        rows_v[j * _CHUNK + r, sl] * 0.25)
                return carry

            lax.fori_loop(0, _CHUNK, scale_row, 0)
        pltpu.sync_copy(rows_v, out_hbm.at[pl.ds(base, b_per_w)])

    return gather_scale


def kernel(users, items, user_table, item_table, edge_user, edge_item):
    B = users.shape[0]
    N, D = user_table.shape
    info = plsc.get_sparse_core_info()
    fn = _make_gather_kernel(B, D, info.num_cores, info.num_subcores)
    utp = jnp.pad(user_table, ((0, 0), (0, _DP - D)))
    itp = jnp.pad(item_table, ((0, 0), (0, _DP - D)))
    out_u = fn(users, utp)
    out_i = fn(items, itp)
    return out_u[:, :D], out_i[:, :D]
